# baseline (speedup 1.0000x reference)
import jax
import jax.numpy as jnp
from jax import lax
from jax.experimental import pallas as pl
from jax.experimental.pallas import tpu as pltpu
from jax.experimental.pallas import tpu_sc as plsc

_NUM_BINS = 1000000
_B = 16384
_NC = 1
_NS = 16
_NW = _NC * _NS
_PER_W = _B // _NW
_NCH = 2
_CH = _PER_W // _NCH
_L = 16


def _mix(h):
    h = h ^ (h >> jnp.uint32(16))
    h = h * jnp.uint32(0x85EBCA6B)
    h = h ^ (h >> jnp.uint32(13))
    h = h * jnp.uint32(0xC2B2AE35)
    h = h ^ (h >> jnp.uint32(16))
    return h


def _hash_range(buf, lo, n):
    for i in range(n // _L):
        a = buf[pl.ds(lo + i * _L, _L)].astype(jnp.uint32)
        b = buf[pl.ds(_PER_W + lo + i * _L, _L)].astype(jnp.uint32)
        h = _mix(a)
        h = h ^ (_mix(b) + jnp.uint32(0x9E3779B9)
                 + (h << jnp.uint32(6)) + (h >> jnp.uint32(2)))
        h = _mix(h)
        buf[pl.ds(2 * _PER_W + lo + i * _L, _L)] = (
            h % jnp.uint32(_NUM_BINS)).astype(jnp.int32)


def _body(f1_hbm, f2_hbm, out_hbm, buf, sem):
    wid = lax.axis_index("s") * _NC + lax.axis_index("c")
    base = wid * _PER_W
    cps = []
    for c in range(_NCH):
        off = c * _CH
        cps.append(pltpu.async_copy(
            f1_hbm.at[pl.ds(base + off, _CH)], buf.at[pl.ds(off, _CH)], sem))
        cps.append(pltpu.async_copy(
            f2_hbm.at[pl.ds(base + off, _CH)],
            buf.at[pl.ds(_PER_W + off, _CH)], sem))
    outs = []
    for c in range(_NCH):
        off = c * _CH
        cps[2 * c].wait()
        cps[2 * c + 1].wait()
        _hash_range(buf, off, _CH)
        outs.append(pltpu.async_copy(
            buf.at[pl.ds(2 * _PER_W + off, _CH)],
            out_hbm.at[pl.ds(base + off, _CH)], sem))
    for o in outs:
        o.wait()


@jax.jit
def kernel(feat1, feat2):
    mesh = plsc.VectorSubcoreMesh(
        core_axis_name="c", subcore_axis_name="s", num_cores=_NC)
    f = pl.kernel(
        _body,
        mesh=mesh,
        out_type=jax.ShapeDtypeStruct((_B,), jnp.int32),
        scratch_types=[
            pltpu.VMEM((3 * _PER_W,), jnp.int32),
            pltpu.SemaphoreType.DMA,
        ],
    )
    return f(feat1, feat2)
